# Initial kernel scaffold; baseline (speedup 1.0000x reference)
import jax
import jax.numpy as jnp
from jax.experimental import pallas as pl


def kernel(x, edge_index, edge_attr, batch, node_W, node_b, edge_W, edge_b, W1, b1, W2, b2, eps, pred_W, pred_b):
    raise NotImplementedError("write your pallas kernel here")


# speedup vs baseline: 1.8945x; 1.8945x over previous
import functools

import jax
import jax.numpy as jnp
from jax import lax
from jax.experimental import pallas as pl
from jax.experimental.pallas import tpu as pltpu
from jax.experimental.pallas import tpu_sc as plsc

N = 10000
NP = 10240
E = 320000
G = 64
EP = 320
H = EP // 2
NL = 5

NC, NS = 2, 16
EW = E // NS
CH = 80
NK = EW // CH
RT = NP // NS
ZB = 128


def _encode_body(x_ref, w_ref, b_ref, out_ref):
    r = jnp.dot(x_ref[...], w_ref[...], preferred_element_type=jnp.float32)
    r = r + b_ref[...]
    out_ref[0] = r[:, :H]
    out_ref[1] = r[:, H:]


def _encode(x, node_Wp, node_bp):
    bm = 1000
    return pl.pallas_call(
        _encode_body,
        grid=(N // bm,),
        in_specs=[
            pl.BlockSpec((bm, 128), lambda i: (i, 0)),
            pl.BlockSpec((128, EP), lambda i: (0, 0)),
            pl.BlockSpec((1, EP), lambda i: (0, 0)),
        ],
        out_specs=pl.BlockSpec((NC, bm, H), lambda i: (0, i, 0)),
        out_shape=jax.ShapeDtypeStruct((NC, NP, H), jnp.float32),
    )(x, node_Wp, node_bp)


def _edge_mlp_body(a_ref, w_ref, b_ref, out_ref):
    r = jnp.dot(a_ref[...], w_ref[...], preferred_element_type=jnp.float32)
    r = jnp.maximum(r + b_ref[...], 0.0)
    out_ref[0] = r[:, :H]
    out_ref[1] = r[:, H:]


def _edge_mlp(edge_attr, Wl, bl):
    bm = 2000
    return pl.pallas_call(
        _edge_mlp_body,
        grid=(E // bm,),
        in_specs=[
            pl.BlockSpec((bm, 16), lambda i: (i, 0)),
            pl.BlockSpec((16, EP), lambda i: (0, 0)),
            pl.BlockSpec((1, EP), lambda i: (0, 0)),
        ],
        out_specs=pl.BlockSpec((NC, bm, H), lambda i: (0, i, 0)),
        out_shape=jax.ShapeDtypeStruct((NC, E, H), jnp.float32),
    )(edge_attr, Wl, bl)


def _mlp_body(last, h_ref, agg_ref, w1_ref, b1_ref, w2_ref, b2_ref, eps_ref,
              out_ref):
    hb = jnp.concatenate([h_ref[0], h_ref[1]], axis=1)
    ab = jnp.concatenate([agg_ref[0], agg_ref[1]], axis=1)
    h2 = (1.0 + eps_ref[0, 0]) * hb + ab
    t = jnp.maximum(
        jnp.dot(h2, w1_ref[...], preferred_element_type=jnp.float32)
        + b1_ref[...], 0.0)
    o = jnp.dot(t, w2_ref[...], preferred_element_type=jnp.float32) + b2_ref[...]
    if not last:
        o = jnp.maximum(o, 0.0)
    out_ref[0] = o[:, :H]
    out_ref[1] = o[:, H:]


def _gin_mlp(h, agg, W1l, b1l, W2l, b2l, epsl, last):
    bm = 1000
    return pl.pallas_call(
        functools.partial(_mlp_body, last),
        grid=(N // bm,),
        in_specs=[
            pl.BlockSpec((NC, bm, H), lambda i: (0, i, 0)),
            pl.BlockSpec((NC, bm, H), lambda i: (0, i, 0)),
            pl.BlockSpec((EP, 640), lambda i: (0, 0)),
            pl.BlockSpec((1, 640), lambda i: (0, 0)),
            pl.BlockSpec((640, EP), lambda i: (0, 0)),
            pl.BlockSpec((1, EP), lambda i: (0, 0)),
            pl.BlockSpec((1, 1), lambda i: (0, 0), memory_space=pltpu.SMEM),
        ],
        out_specs=pl.BlockSpec((NC, bm, H), lambda i: (0, i, 0)),
        out_shape=jax.ShapeDtypeStruct((NC, NP, H), jnp.float32),
    )(h, agg, W1l, b1l, W2l, b2l, epsl)


def _pool_body(h_ref, batch_ref, pw_ref, pb_ref, pred_ref, norm_ref,
               acc_ref, cnt_ref):
    i = pl.program_id(0)

    @pl.when(i == 0)
    def _init():
        acc_ref[...] = jnp.zeros_like(acc_ref)
        cnt_ref[...] = jnp.zeros_like(cnt_ref)

    hb = jnp.concatenate([h_ref[0], h_ref[1]], axis=1)
    bm = hb.shape[0]
    gids = jax.lax.broadcasted_iota(jnp.int32, (bm, G), 1)
    onehot = (batch_ref[0, 0][:, None] == gids).astype(jnp.float32)
    acc_ref[...] += jax.lax.dot_general(
        onehot, hb, (((0,), (0,)), ((), ())),
        preferred_element_type=jnp.float32)
    cnt_ref[...] += jax.lax.dot_general(
        onehot, jnp.ones((bm, 128), jnp.float32), (((0,), (0,)), ((), ())),
        preferred_element_type=jnp.float32)

    @pl.when(i == pl.num_programs(0) - 1)
    def _done():
        cnt = jnp.maximum(cnt_ref[:, 0:1], 1.0)
        hg = acc_ref[...] / cnt
        logit = jnp.dot(hg, pw_ref[...], preferred_element_type=jnp.float32)
        p = logit[:, 0] + pb_ref[0, 0]
        pred_ref[0] = p
        norm_ref[0] = 2.0 * ((p - (-10.0)) / 6.0) - 1.0


def _pool_head(h, batch2d, pred_Wp, pred_b2d):
    bm = 1000
    return pl.pallas_call(
        _pool_body,
        grid=(N // bm,),
        in_specs=[
            pl.BlockSpec((NC, bm, H), lambda i: (0, i, 0)),
            pl.BlockSpec((1, 1, bm), lambda i: (i, 0, 0)),
            pl.BlockSpec((EP, 128), lambda i: (0, 0)),
            pl.BlockSpec((1, 1), lambda i: (0, 0), memory_space=pltpu.SMEM),
        ],
        out_specs=[
            pl.BlockSpec((1, G), lambda i: (0, 0)),
            pl.BlockSpec((1, G), lambda i: (0, 0)),
        ],
        out_shape=[
            jax.ShapeDtypeStruct((1, G), jnp.float32),
            jax.ShapeDtypeStruct((1, G), jnp.float32),
        ],
        scratch_shapes=[
            pltpu.VMEM((G, EP), jnp.float32),
            pltpu.VMEM((G, 128), jnp.float32),
        ],
    )(h, batch2d, pred_Wp, pred_b2d)


def _sc_layer_body(h_hbm, e_hbm, src_hbm, dst_hbm, out_hbm,
                   acc, src_v, dst_v, msg_v, rows_v, sem):
    c = lax.axis_index("c")
    s = lax.axis_index("s")

    def _zrow(i, carry):
        for v in range(H // 16):
            rows_v[i, pl.ds(v * 16, 16)] = jnp.zeros((16,), jnp.float32)
        return carry
    lax.fori_loop(0, CH, _zrow, None)
    for j in range(RT // CH):
        pltpu.sync_copy(rows_v, acc.at[pl.ds(s * RT + j * CH, CH)])

    plsc.subcore_barrier()

    def _chunk(k, carry):
        pltpu.sync_copy(src_hbm.at[s, k], src_v)
        pltpu.sync_copy(dst_hbm.at[s, k], dst_v)
        pltpu.sync_copy(e_hbm.at[c, pl.ds(s * EW + k * CH, CH)], msg_v)
        pltpu.async_copy(h_hbm.at[c].at[src_v], rows_v, sem).wait()

        def _edge(j, carry2):
            for v in range(H // 16):
                sl = pl.ds(v * 16, 16)
                m = rows_v[j, sl] + msg_v[j, sl]
                msg_v[j, sl] = jnp.maximum(m, 0.0)
            return carry2
        lax.fori_loop(0, CH, _edge, None)

        pltpu.sync_copy(msg_v, acc.at[dst_v], add=True)
        return carry
    lax.fori_loop(0, NK, _chunk, None)

    plsc.subcore_barrier()

    pltpu.sync_copy(acc.at[pl.ds(s * RT, RT)],
                    out_hbm.at[c].at[pl.ds(s * RT, RT)])


def _sc_layer(h, e, src_t, dst_t):
    mesh = plsc.VectorSubcoreMesh(core_axis_name="c", subcore_axis_name="s",
                                  num_cores=NC, num_subcores=NS)
    f = pl.kernel(
        _sc_layer_body,
        out_type=jax.ShapeDtypeStruct((NC, NP, H), jnp.float32),
        mesh=mesh,
        compiler_params=pltpu.CompilerParams(use_tc_tiling_on_sc=False),
        scratch_types=[
            pltpu.VMEM_SHARED((NP, H), jnp.float32),
            pltpu.VMEM((CH,), jnp.int32),
            pltpu.VMEM((CH,), jnp.int32),
            pltpu.VMEM((CH, H), jnp.float32),
            pltpu.VMEM((CH, H), jnp.float32),
            pltpu.SemaphoreType.DMA,
        ],
    )
    return f(h, e, src_t, dst_t)


def kernel(x, edge_index, edge_attr, batch, node_W, node_b, edge_W, edge_b,
           W1, b1, W2, b2, eps, pred_W, pred_b):
    f32 = jnp.float32
    node_Wp = jnp.zeros((128, EP), f32).at[:, :300].set(node_W)
    node_bp = jnp.zeros((1, EP), f32).at[0, :300].set(node_b)
    edge_Wp = jnp.zeros((NL, 16, EP), f32).at[:, :, :300].set(edge_W)
    edge_bp = jnp.zeros((NL, 1, EP), f32).at[:, 0, :300].set(edge_b)
    W1p = jnp.zeros((NL, EP, 640), f32).at[:, :300, :600].set(W1)
    b1p = jnp.zeros((NL, 1, 640), f32).at[:, 0, :600].set(b1)
    W2p = jnp.zeros((NL, 640, EP), f32).at[:, :600, :300].set(W2)
    b2p = jnp.zeros((NL, 1, EP), f32).at[:, 0, :300].set(b2)
    pred_Wp = jnp.zeros((EP, 128), f32).at[:300, 0].set(pred_W[:, 0])
    pred_b2d = pred_b.reshape(1, 1)
    eps2d = eps.reshape(NL, 1, 1)

    src_t = edge_index[0].reshape(NS, NK, CH)
    dst_t = edge_index[1].reshape(NS, NK, CH)
    batch2d = batch.reshape(N // 1000, 1, 1000)

    h = _encode(x, node_Wp, node_bp)
    for l in range(NL):
        e = _edge_mlp(edge_attr, edge_Wp[l], edge_bp[l])
        agg = _sc_layer(h, e, src_t, dst_t)
        h = _gin_mlp(h, agg, W1p[l], b1p[l], W2p[l], b2p[l], eps2d[l],
                     last=(l == NL - 1))

    pred2d, norm2d = _pool_head(h, batch2d, pred_Wp, pred_b2d)
    return (pred2d[0], norm2d[0])
